# Initial kernel scaffold; baseline (speedup 1.0000x reference)
import jax
import jax.numpy as jnp
from jax.experimental import pallas as pl


def kernel(x, edge_index, W_in, b_in, g_in, be_in, Wn0, bb0, Wr0, g0, be0, Wn1, bb1, Wr1, g1, be1):
    raise NotImplementedError("write your pallas kernel here")


# speedup vs baseline: 6.4110x; 6.4110x over previous
import functools

import jax
import jax.numpy as jnp
from jax import lax
from jax.experimental import pallas as pl
from jax.experimental.pallas import tpu as pltpu
from jax.experimental.pallas import tpu_sc as plsc

N = 10000
E = 320000
D = 128
EPS = 1e-5

NC = 2
NS = 16
NW = NC * NS

CH = 40
EPW = E // NW
NCHUNK = EPW // CH
RPT = 640
NP = NS * RPT
ZR = RPT // 5
CW = 16

_mesh = plsc.VectorSubcoreMesh(core_axis_name="c", subcore_axis_name="s")


@functools.partial(
    pl.kernel,
    out_type=(
        jax.ShapeDtypeStruct((NC, NP, D), jnp.float32),
        jax.ShapeDtypeStruct((NC, NP, CW), jnp.float32),
    ),
    mesh=_mesh,
    scratch_types=[
        pltpu.VMEM((EPW,), jnp.int32),
        pltpu.VMEM((NCHUNK, CH), jnp.int32),
        pltpu.VMEM((CH, D), jnp.float32),
        pltpu.VMEM((CH, D), jnp.float32),
        pltpu.VMEM((CH, CW), jnp.float32),
        pltpu.VMEM_SHARED((NP, D), jnp.float32),
        pltpu.VMEM_SHARED((NP, CW), jnp.float32),
        pltpu.SemaphoreType.DMA,
        pltpu.SemaphoreType.DMA,
    ],
    compiler_params=pltpu.CompilerParams(use_tc_tiling_on_sc=False),
)
def _sc_segment_sum(h_hbm, src_hbm, dst_hbm, zf_hbm, zc_hbm, on_hbm,
                    agg_out, cnt_out,
                    src_v, dst_v, rows0, rows1, ones_v,
                    acc_sh, cnt_sh, sem0, sem1):
    c = lax.axis_index("c")
    s = lax.axis_index("s")
    wid = s * NC + c

    pltpu.sync_copy(on_hbm, ones_v)
    pltpu.sync_copy(zf_hbm, acc_sh.at[pl.ds(s * RPT, RPT)])
    pltpu.sync_copy(zc_hbm, cnt_sh.at[pl.ds(s * RPT, RPT)])
    plsc.subcore_barrier()

    pltpu.sync_copy(src_hbm.at[pl.ds(wid * EPW, EPW)], src_v)
    pltpu.sync_copy(dst_hbm.at[wid], dst_v)

    pltpu.async_copy(h_hbm.at[src_v.at[pl.ds(0, CH)]], rows0, sem0)

    def body(j, carry):
        b0 = j * (2 * CH)
        pltpu.make_async_copy(h_hbm.at[src_v.at[pl.ds(b0, CH)]], rows0, sem0).wait()
        pltpu.async_copy(h_hbm.at[src_v.at[pl.ds(b0 + CH, CH)]], rows1, sem1)
        pltpu.sync_copy(rows0, acc_sh.at[dst_v.at[2 * j]], add=True)
        pltpu.sync_copy(ones_v, cnt_sh.at[dst_v.at[2 * j]], add=True)
        pltpu.make_async_copy(h_hbm.at[src_v.at[pl.ds(b0 + CH, CH)]], rows1, sem1).wait()

        @pl.when(j < (NCHUNK // 2) - 1)
        def _():
            pltpu.async_copy(h_hbm.at[src_v.at[pl.ds(b0 + 2 * CH, CH)]], rows0, sem0)

        pltpu.sync_copy(rows1, acc_sh.at[dst_v.at[2 * j + 1]], add=True)
        pltpu.sync_copy(ones_v, cnt_sh.at[dst_v.at[2 * j + 1]], add=True)
        return carry

    lax.fori_loop(0, NCHUNK // 2, body, 0)
    plsc.subcore_barrier()

    pltpu.sync_copy(acc_sh.at[pl.ds(s * RPT, RPT)], agg_out.at[c, pl.ds(s * RPT, RPT)])
    pltpu.sync_copy(cnt_sh.at[pl.ds(s * RPT, RPT)], cnt_out.at[c, pl.ds(s * RPT, RPT)])


BR = 1000


def _in_body(x_ref, w_ref, b_ref, s_ref, t_ref, o_ref):
    h = jnp.dot(x_ref[...], w_ref[...], preferred_element_type=jnp.float32)
    h = (h + b_ref[...]) * s_ref[...] + t_ref[...]
    o_ref[...] = jnp.maximum(h, 0.0)


def _tc_input_layer(x, W, b, s, t):
    return pl.pallas_call(
        _in_body,
        grid=(N // BR,),
        in_specs=[
            pl.BlockSpec((BR, D), lambda i: (i, 0)),
            pl.BlockSpec((D, D), lambda i: (0, 0)),
            pl.BlockSpec((1, D), lambda i: (0, 0)),
            pl.BlockSpec((1, D), lambda i: (0, 0)),
            pl.BlockSpec((1, D), lambda i: (0, 0)),
        ],
        out_specs=pl.BlockSpec((BR, D), lambda i: (i, 0)),
        out_shape=jax.ShapeDtypeStruct((N, D), jnp.float32),
    )(x, W, b, s, t)


def _sage_body(h_ref, agg_ref, cnt_ref, wn_ref, wr_ref, bb_ref, s_ref, t_ref, o_ref):
    a = agg_ref[0] + agg_ref[1]
    cnt = cnt_ref[0] + cnt_ref[1]
    inv = 1.0 / jnp.maximum(cnt[:, :1], 1.0)
    a = a * inv
    z = jnp.dot(a, wn_ref[...], preferred_element_type=jnp.float32)
    z = z + bb_ref[...]
    z = z + jnp.dot(h_ref[...], wr_ref[...], preferred_element_type=jnp.float32)
    z = z * s_ref[...] + t_ref[...]
    o_ref[...] = h_ref[...] + jnp.maximum(z, 0.0)


def _tc_sage_layer(h, agg, cnt, Wn, Wr, bb, s, t):
    return pl.pallas_call(
        _sage_body,
        grid=(N // BR,),
        in_specs=[
            pl.BlockSpec((BR, D), lambda i: (i, 0)),
            pl.BlockSpec((NC, BR, D), lambda i: (0, i, 0)),
            pl.BlockSpec((NC, BR, CW), lambda i: (0, i, 0)),
            pl.BlockSpec((D, D), lambda i: (0, 0)),
            pl.BlockSpec((D, D), lambda i: (0, 0)),
            pl.BlockSpec((1, D), lambda i: (0, 0)),
            pl.BlockSpec((1, D), lambda i: (0, 0)),
            pl.BlockSpec((1, D), lambda i: (0, 0)),
        ],
        out_specs=pl.BlockSpec((BR, D), lambda i: (i, 0)),
        out_shape=jax.ShapeDtypeStruct((N, D), jnp.float32),
    )(h, agg, cnt, Wn, Wr, bb, s, t)


def kernel(x, edge_index, W_in, b_in, g_in, be_in,
           Wn0, bb0, Wr0, g0, be0, Wn1, bb1, Wr1, g1, be1):
    bn_scale = 1.0 / jnp.sqrt(jnp.float32(1.0 + EPS))
    row = lambda v: v.reshape(1, D)
    s_in, t_in = row(g_in * bn_scale), row(be_in)
    s0, t0 = row(g0 * bn_scale), row(be0)
    s1, t1 = row(g1 * bn_scale), row(be1)

    src = edge_index[0]
    dst = edge_index[1].reshape(NW, NCHUNK, CH)
    zf = jnp.zeros((RPT, D), jnp.float32)
    zc = jnp.zeros((RPT, CW), jnp.float32)
    on = jnp.ones((CH, CW), jnp.float32)

    h = _tc_input_layer(x, W_in, row(b_in), s_in, t_in)

    agg0, cnt = _sc_segment_sum(h, src, dst, zf, zc, on)
    h = _tc_sage_layer(h, agg0, cnt, Wn0, Wr0, row(bb0), s0, t0)

    agg1, _ = _sc_segment_sum(h, src, dst, zf, zc, on)
    h = _tc_sage_layer(h, agg1, cnt, Wn1, Wr1, row(bb1), s1, t1)
    return h
